# baseline (speedup 1.0000x reference)
import functools

import jax
import jax.numpy as jnp
from jax import lax
from jax.experimental import pallas as pl
from jax.experimental.pallas import tpu as pltpu
from jax.experimental.pallas import tpu_sc as plsc

B = 1024
H = 128
IN = 128
L = 2
S = 100
NROWS = (S + 1) * B * L

_NC = 2
_NS = 16
_NW = _NC * _NS
_BPW = B // _NW
_LANES = 16


def _sc_gather_body(hflat, cflat, posr, h0_out, h1_out, c0_out, c1_out,
                    idx_v, q0_v, q1_v, h0_v, h1_v, c0_v, c1_v,
                    sem_h0, sem_h1, sem_c0, sem_c1,
                    sem_st0, sem_st1, sem_st2, sem_st3):
    wid = lax.axis_index("s") * _NC + lax.axis_index("c")
    base = wid * _BPW
    pltpu.sync_copy(posr.at[pl.ds(base, _BPW)], idx_v)
    for j in range(_BPW // _LANES):
        p = idx_v[pl.ds(j * _LANES, _LANES)]
        lane = lax.iota(jnp.int32, _LANES)
        rid = p * B + (base + j * _LANES) + lane
        q0_v[pl.ds(j * _LANES, _LANES)] = 2 * rid
        q1_v[pl.ds(j * _LANES, _LANES)] = 2 * rid + 1
    cp_h0 = pltpu.async_copy(hflat.at[q0_v], h0_v, sem_h0)
    cp_h1 = pltpu.async_copy(hflat.at[q1_v], h1_v, sem_h1)
    cp_c0 = pltpu.async_copy(cflat.at[q0_v], c0_v, sem_c0)
    cp_c1 = pltpu.async_copy(cflat.at[q1_v], c1_v, sem_c1)
    cp_h0.wait()
    st_h0 = pltpu.async_copy(h0_v, h0_out.at[pl.ds(base, _BPW)], sem_st0)
    cp_h1.wait()
    st_h1 = pltpu.async_copy(h1_v, h1_out.at[pl.ds(base, _BPW)], sem_st1)
    cp_c0.wait()
    st_c0 = pltpu.async_copy(c0_v, c0_out.at[pl.ds(base, _BPW)], sem_st2)
    cp_c1.wait()
    st_c1 = pltpu.async_copy(c1_v, c1_out.at[pl.ds(base, _BPW)], sem_st3)
    st_h0.wait()
    st_h1.wait()
    st_c0.wait()
    st_c1.wait()


@functools.cache
def _sc_gather():
    row = jax.ShapeDtypeStruct((B, H), jnp.float32)
    return pl.kernel(
        _sc_gather_body,
        out_type=[row, row, row, row],
        mesh=plsc.VectorSubcoreMesh(core_axis_name="c", subcore_axis_name="s",
                                    num_cores=_NC, num_subcores=_NS),
        scratch_types=[
            pltpu.VMEM((_BPW,), jnp.int32),
            pltpu.VMEM((_BPW,), jnp.int32),
            pltpu.VMEM((_BPW,), jnp.int32),
            pltpu.VMEM((_BPW, H), jnp.float32),
            pltpu.VMEM((_BPW, H), jnp.float32),
            pltpu.VMEM((_BPW, H), jnp.float32),
            pltpu.VMEM((_BPW, H), jnp.float32),
            pltpu.SemaphoreType.DMA,
            pltpu.SemaphoreType.DMA,
            pltpu.SemaphoreType.DMA,
            pltpu.SemaphoreType.DMA,
            pltpu.SemaphoreType.DMA,
            pltpu.SemaphoreType.DMA,
            pltpu.SemaphoreType.DMA,
            pltpu.SemaphoreType.DMA,
        ],
    )


def _dot_t(a, b):
    return lax.dot_general(a, b, (((1,), (1,)), ((), ())),
                           precision=lax.Precision.DEFAULT,
                           preferred_element_type=jnp.float32)


def _lstm_body(x_ref, h0_ref, h1_ref, c0_ref, c1_ref, op_ref,
               wih0_ref, whh0_ref, bih0_ref, bhh0_ref,
               wih1_ref, whh1_ref, bih1_ref, bhh1_ref,
               out_ref):
    def cell(x, h, c, wih, whh, bih, bhh):
        gates = _dot_t(x, wih) + _dot_t(h, whh) + (bih + bhh)
        i = gates[:, 0:H]
        f = gates[:, H:2 * H]
        g = gates[:, 2 * H:3 * H]
        o = gates[:, 3 * H:4 * H]
        c2 = jax.nn.sigmoid(f) * c + jax.nn.sigmoid(i) * jnp.tanh(g)
        h2 = jax.nn.sigmoid(o) * jnp.tanh(c2)
        return h2

    h1 = h1_ref[...]
    h2_0 = cell(x_ref[...], h0_ref[...], c0_ref[...],
                wih0_ref[...], whh0_ref[...], bih0_ref[...], bhh0_ref[...])
    h2_1 = cell(h2_0, h1, c1_ref[...],
                wih1_ref[...], whh1_ref[...], bih1_ref[...], bhh1_ref[...])

    out_ref[...] = jnp.where(op_ref[...] > 0, h2_1, h1)


_BB = 256
_NBB = B // _BB

_row_spec = pl.BlockSpec((_BB, H), lambda i: (i, 0))
_full2 = lambda r, c: pl.BlockSpec((r, c), lambda i: (0, 0))

_lstm = pl.pallas_call(
    _lstm_body,
    grid=(_NBB,),
    in_specs=[
        _row_spec,
        _row_spec, _row_spec, _row_spec, _row_spec,
        pl.BlockSpec((_BB, 1), lambda i: (i, 0)),
        _full2(4 * H, IN), _full2(4 * H, H),
        _full2(1, 4 * H), _full2(1, 4 * H),
        _full2(4 * H, H), _full2(4 * H, H),
        _full2(1, 4 * H), _full2(1, 4 * H),
    ],
    out_specs=_row_spec,
    out_shape=jax.ShapeDtypeStruct((B, H), jnp.float32),
)


def kernel(input, hidden_stack, cell_stack, op, pos,
           W_ih0, W_hh0, b_ih0, b_hh0, W_ih1, W_hh1, b_ih1, b_hh1):
    hflat = jnp.transpose(hidden_stack, (0, 1, 3, 2)).reshape(NROWS, H)
    cflat = jnp.transpose(cell_stack, (0, 1, 3, 2)).reshape(NROWS, H)
    pos32 = pos.astype(jnp.int32)
    h0, h1, c0, c1 = _sc_gather()(hflat, cflat, pos32)

    return _lstm(input, h0, h1, c0, c1, op.reshape(B, 1),
                 W_ih0, W_hh0, b_ih0.reshape(1, 4 * H), b_hh0.reshape(1, 4 * H),
                 W_ih1, W_hh1, b_ih1.reshape(1, 4 * H), b_hh1.reshape(1, 4 * H))


# speedup vs baseline: 1.0218x; 1.0218x over previous
import functools

import jax
import jax.numpy as jnp
from jax import lax
from jax.experimental import pallas as pl
from jax.experimental.pallas import tpu as pltpu
from jax.experimental.pallas import tpu_sc as plsc

B = 1024
H = 128
IN = 128
L = 2
S = 100
NROWS = (S + 1) * B * L

_NC = 2
_NS = 16
_NW = _NC * _NS
_BPW = B // _NW
_LANES = 16


def _sc_gather_body(hflat, cflat, posr, h0_out, h1_out, c0_out, c1_out,
                    idx_v, q0_v, q1_v, h0_v, h1_v, c0_v, c1_v,
                    sem_h0, sem_h1, sem_c0, sem_c1,
                    sem_st0, sem_st1, sem_st2, sem_st3):
    wid = lax.axis_index("s") * _NC + lax.axis_index("c")
    base = wid * _BPW
    pltpu.sync_copy(posr.at[pl.ds(base, _BPW)], idx_v)
    for j in range(_BPW // _LANES):
        p = idx_v[pl.ds(j * _LANES, _LANES)]
        lane = lax.iota(jnp.int32, _LANES)
        rid = p * B + (base + j * _LANES) + lane
        q0_v[pl.ds(j * _LANES, _LANES)] = 2 * rid
        q1_v[pl.ds(j * _LANES, _LANES)] = 2 * rid + 1
    cp_h0 = pltpu.async_copy(hflat.at[q0_v], h0_v, sem_h0)
    cp_h1 = pltpu.async_copy(hflat.at[q1_v], h1_v, sem_h1)
    cp_c0 = pltpu.async_copy(cflat.at[q0_v], c0_v, sem_c0)
    cp_c1 = pltpu.async_copy(cflat.at[q1_v], c1_v, sem_c1)
    cp_h0.wait()
    st_h0 = pltpu.async_copy(h0_v, h0_out.at[pl.ds(base, _BPW)], sem_st0)
    cp_h1.wait()
    st_h1 = pltpu.async_copy(h1_v, h1_out.at[pl.ds(base, _BPW)], sem_st1)
    cp_c0.wait()
    st_c0 = pltpu.async_copy(c0_v, c0_out.at[pl.ds(base, _BPW)], sem_st2)
    cp_c1.wait()
    st_c1 = pltpu.async_copy(c1_v, c1_out.at[pl.ds(base, _BPW)], sem_st3)
    st_h0.wait()
    st_h1.wait()
    st_c0.wait()
    st_c1.wait()


@functools.cache
def _sc_gather():
    row = jax.ShapeDtypeStruct((B, H), jnp.float32)
    return pl.kernel(
        _sc_gather_body,
        out_type=[row, row, row, row],
        mesh=plsc.VectorSubcoreMesh(core_axis_name="c", subcore_axis_name="s",
                                    num_cores=_NC, num_subcores=_NS),
        scratch_types=[
            pltpu.VMEM((_BPW,), jnp.int32),
            pltpu.VMEM((_BPW,), jnp.int32),
            pltpu.VMEM((_BPW,), jnp.int32),
            pltpu.VMEM((_BPW, H), jnp.float32),
            pltpu.VMEM((_BPW, H), jnp.float32),
            pltpu.VMEM((_BPW, H), jnp.float32),
            pltpu.VMEM((_BPW, H), jnp.float32),
            pltpu.SemaphoreType.DMA,
            pltpu.SemaphoreType.DMA,
            pltpu.SemaphoreType.DMA,
            pltpu.SemaphoreType.DMA,
            pltpu.SemaphoreType.DMA,
            pltpu.SemaphoreType.DMA,
            pltpu.SemaphoreType.DMA,
            pltpu.SemaphoreType.DMA,
        ],
    )


def _dot_t(a, b):
    return lax.dot_general(a, b, (((1,), (1,)), ((), ())),
                           precision=lax.Precision.DEFAULT,
                           preferred_element_type=jnp.float32)


def _lstm_body(x_ref, h0_ref, h1_ref, c0_ref, c1_ref, op_ref,
               wih0_ref, whh0_ref, bih0_ref, bhh0_ref,
               wih1_ref, whh1_ref, bih1_ref, bhh1_ref,
               out_ref):
    def cell(x, h, c, wih, whh, bih, bhh):
        gates = _dot_t(x, wih) + _dot_t(h, whh) + (bih + bhh)
        i = gates[:, 0:H]
        f = gates[:, H:2 * H]
        g = gates[:, 2 * H:3 * H]
        o = gates[:, 3 * H:4 * H]
        c2 = jax.nn.sigmoid(f) * c + jax.nn.sigmoid(i) * jnp.tanh(g)
        h2 = jax.nn.sigmoid(o) * jnp.tanh(c2)
        return h2

    h1 = h1_ref[...]
    h2_0 = cell(x_ref[...], h0_ref[...], c0_ref[...],
                wih0_ref[...], whh0_ref[...], bih0_ref[...], bhh0_ref[...])
    h2_1 = cell(h2_0, h1, c1_ref[...],
                wih1_ref[...], whh1_ref[...], bih1_ref[...], bhh1_ref[...])

    out_ref[...] = jnp.where(op_ref[...] > 0, h2_1, h1)


_lstm = pl.pallas_call(
    _lstm_body,
    out_shape=jax.ShapeDtypeStruct((B, H), jnp.float32),
)


def kernel(input, hidden_stack, cell_stack, op, pos,
           W_ih0, W_hh0, b_ih0, b_hh0, W_ih1, W_hh1, b_ih1, b_hh1):
    hflat = jnp.transpose(hidden_stack, (0, 1, 3, 2)).reshape(NROWS, H)
    cflat = jnp.transpose(cell_stack, (0, 1, 3, 2)).reshape(NROWS, H)
    pos32 = pos.astype(jnp.int32)
    h0, h1, c0, c1 = _sc_gather()(hflat, cflat, pos32)

    return _lstm(input, h0, h1, c0, c1, op.reshape(B, 1),
                 W_ih0, W_hh0, b_ih0.reshape(1, 4 * H), b_hh0.reshape(1, 4 * H),
                 W_ih1, W_hh1, b_ih1.reshape(1, 4 * H), b_hh1.reshape(1, 4 * H))
